# baseline (speedup 1.0000x reference)
import functools

import jax
import jax.numpy as jnp
from jax import lax
from jax.experimental import pallas as pl
from jax.experimental.pallas import tpu as pltpu
from jax.experimental.pallas import tpu_sc as plsc

_MOVING_RATE = 0.999
_TILE = 2048
_NC, _NS = 2, 16
_NW = _NC * _NS


def _norm_rows(x):
    n = jnp.sqrt(jnp.sum(x * x, axis=1, keepdims=True))
    return x / jnp.maximum(n, 1e-12)


def _dot(a, b, dims):
    return jax.lax.dot_general(
        a, b, (dims, ((), ())),
        preferred_element_type=jnp.float32,
        precision=jax.lax.Precision.DEFAULT)


def _tc1_kernel(key_ref, mem_ref, ind_ref):
    _, B, D = key_ref.shape
    E = mem_ref.shape[1]
    T = min(_TILE, E)
    NT = E // T

    xn = _norm_rows(key_ref[0])

    def pass_a(t, carry):
        run_max, run_arg = carry
        m_t = mem_ref[0, pl.ds(t * T, T), :]
        mn_t = _norm_rows(m_t)
        s = _dot(xn, mn_t, ((1,), (1,)))
        tmax = jnp.max(s, axis=1)
        targ = jnp.argmax(s, axis=1).astype(jnp.int32) + t * T
        upd = tmax > run_max
        return (jnp.where(upd, tmax, run_max),
                jnp.where(upd, targ, run_arg))

    neg = jnp.full((B,), -jnp.inf, jnp.float32)
    _, embed_ind = jax.lax.fori_loop(
        0, NT, pass_a, (neg, jnp.zeros((B,), jnp.int32)))
    ind_ref[0, 0, :] = embed_ind


def _tc2_kernel(key_ref, value_ref, ind_ref, mem_ref, memout_ref, ind2_ref):
    _, B, D = key_ref.shape
    E = mem_ref.shape[1]
    T = min(_TILE, E)
    NT = E // T
    slot = pl.program_id(0)

    xn = _norm_rows(key_ref[0])
    v = value_ref[0]
    embed_ind = ind_ref[0, 0, :]

    def pass_b(t, carry):
        run_max2, run_arg2 = carry
        m_t = mem_ref[0, pl.ds(t * T, T), :]
        col = t * T + jax.lax.broadcasted_iota(jnp.int32, (B, T), 1)
        oneh = (embed_ind[:, None] == col).astype(jnp.float32)
        counts = jnp.sum(oneh, axis=0)
        esum = _dot(oneh, v, ((0,), (0,)))
        new_m = (m_t * _MOVING_RATE
                 + (esum / (counts[:, None] + 1e-06)) * (1.0 - _MOVING_RATE))
        memout_ref[0, pl.ds(t * T, T), :] = new_m
        mn2 = _norm_rows(new_m)
        s2 = _dot(xn, mn2, ((1,), (1,)))
        tmax2 = jnp.max(s2, axis=1)
        targ2 = jnp.argmax(s2, axis=1).astype(jnp.int32) + t * T
        upd = tmax2 > run_max2
        return (jnp.where(upd, tmax2, run_max2),
                jnp.where(upd, targ2, run_arg2))

    neg = jnp.full((B,), -jnp.inf, jnp.float32)
    _, run_arg2 = jax.lax.fori_loop(
        0, NT, pass_b, (neg, jnp.zeros((B,), jnp.int32)))
    ind2_ref[0, 0, :] = run_arg2 + slot * E


def _sc_gather(table, ind2_flat, v0):
    SB = ind2_flat.shape[0]
    D = table.shape[1]
    R = SB // _NW

    def body(table_ref, ind2_ref, v0_ref, out_ref,
             idx_v, rows_v, v0_v, sem):
        wid = lax.axis_index("s") * _NC + lax.axis_index("c")
        base = wid * R
        b0 = (wid % (v0_ref.shape[0] // R)) * R
        pltpu.sync_copy(ind2_ref.at[pl.ds(base, R)], idx_v)
        pltpu.async_copy(table_ref.at[idx_v], rows_v, sem).wait()
        pltpu.sync_copy(v0_ref.at[pl.ds(b0, R)], v0_v)

        def add_row(i, _):
            for j in range(D // 16):
                sl = pl.ds(j * 16, 16)
                rows_v[i, sl] = rows_v[i, sl] + v0_v[i, sl]
            return 0

        lax.fori_loop(0, R, add_row, 0)
        pltpu.sync_copy(rows_v, out_ref.at[pl.ds(base, R)])

    return pl.kernel(
        body,
        out_type=jax.ShapeDtypeStruct((SB, D), jnp.float32),
        mesh=plsc.VectorSubcoreMesh(
            core_axis_name="c", subcore_axis_name="s"),
        scratch_types=[
            pltpu.VMEM((R,), jnp.int32),
            pltpu.VMEM((R, D), jnp.float32),
            pltpu.VMEM((R, D), jnp.float32),
            pltpu.SemaphoreType.DMA,
        ],
        compiler_params=pltpu.CompilerParams(use_tc_tiling_on_sc=False),
    )(table, ind2_flat, v0)


def kernel(key, value, memory):
    B, S, D = key.shape
    E = memory.shape[1]
    key_t = key.transpose(1, 0, 2)
    value_t = value.transpose(1, 0, 2)
    v0 = value[:, 0, :]

    ind = pl.pallas_call(
        _tc1_kernel,
        grid=(S,),
        in_specs=[
            pl.BlockSpec((1, B, D), lambda i: (i, 0, 0)),
            pl.BlockSpec((1, E, D), lambda i: (i, 0, 0)),
        ],
        out_specs=pl.BlockSpec((1, 1, B), lambda i: (i, 0, 0)),
        out_shape=jax.ShapeDtypeStruct((S, 1, B), jnp.int32),
        compiler_params=pltpu.CompilerParams(
            dimension_semantics=("arbitrary",)),
    )(key_t, memory)

    mem, ind2 = pl.pallas_call(
        _tc2_kernel,
        grid=(S,),
        in_specs=[
            pl.BlockSpec((1, B, D), lambda i: (i, 0, 0)),
            pl.BlockSpec((1, B, D), lambda i: (i, 0, 0)),
            pl.BlockSpec((1, 1, B), lambda i: (i, 0, 0)),
            pl.BlockSpec((1, E, D), lambda i: (i, 0, 0)),
        ],
        out_specs=[
            pl.BlockSpec((1, E, D), lambda i: (i, 0, 0)),
            pl.BlockSpec((1, 1, B), lambda i: (i, 0, 0)),
        ],
        out_shape=[
            jax.ShapeDtypeStruct((S, E, D), jnp.float32),
            jax.ShapeDtypeStruct((S, 1, B), jnp.int32),
        ],
        compiler_params=pltpu.CompilerParams(
            dimension_semantics=("arbitrary",)),
    )(key_t, value_t, ind, memory)

    out_flat = _sc_gather(
        mem.reshape(S * E, D), ind2.reshape(S * B), v0)
    out = out_flat.reshape(S, B, D).transpose(1, 0, 2)

    return (key, value, out, mem)
